# baseline (speedup 1.0000x reference)
import functools

import jax
import jax.numpy as jnp
from jax import lax
from jax.experimental import pallas as pl
from jax.experimental.pallas import tpu as pltpu
from jax.experimental.pallas import tpu_sc as plsc

F32 = jnp.float32
I32 = jnp.int32
NEG_INF = float("-inf")

_INTERP = False
_USE_JNP_SPARSE = False

NW = 32
_EDGE_BLK = 2000
_NODE_BLK = 1000
_GBLK = 16


def _lnsilu(y, g, b):
    mu = jnp.mean(y, axis=-1, keepdims=True)
    yc = y - mu
    var = jnp.mean(yc * yc, axis=-1, keepdims=True)
    z = yc * (lax.rsqrt(var + 1e-5) * g) + b
    zh = 0.5 * z
    return zh + zh * jnp.tanh(zh)


def _sigmoid(x):
    return 0.5 * (jnp.tanh(0.5 * x) + 1.0)


def _pack_bf(xb):
    lo = lax.bitcast_convert_type(xb[:, :64], jnp.uint16).astype(jnp.uint32)
    hi = lax.bitcast_convert_type(xb[:, 64:], jnp.uint16).astype(jnp.uint32)
    return lax.bitcast_convert_type(lo | (hi << 16), jnp.int32)


def _unpack_bf(xi):
    u = lax.bitcast_convert_type(xi, jnp.uint32)
    lo = lax.bitcast_convert_type((u & 0xFFFF).astype(jnp.uint16),
                                  jnp.bfloat16)
    hi = lax.bitcast_convert_type(
        lax.shift_right_logical(u, jnp.uint32(16)).astype(jnp.uint16),
        jnp.bfloat16)
    return jnp.concatenate([lo, hi], axis=1)


def _full_spec(arr):
    nd = arr.ndim
    return pl.BlockSpec(arr.shape, lambda i, _nd=nd: (0,) * _nd)


def _embed_body(x_ref, w_ref, b_ref, o_ref):
    o_ref[...] = jnp.dot(x_ref[...], w_ref[...],
                         preferred_element_type=F32) + b_ref[...]


def _tc_embed(x, w, b):
    n, d = x.shape
    h = w.shape[1]
    grid = n // _NODE_BLK
    return pl.pallas_call(
        _embed_body,
        grid=(grid,),
        in_specs=[pl.BlockSpec((_NODE_BLK, d), lambda i: (i, 0)),
                  _full_spec(w), _full_spec(b)],
        out_specs=pl.BlockSpec((_NODE_BLK, h), lambda i: (i, 0)),
        out_shape=jax.ShapeDtypeStruct((n, h), F32),
        interpret=_INTERP,
    )(x, w, b)


def _edge_body(gd_ref, gs_ref, eat_ref, we_ref, be_ref, ge_ref, bbe_ref,
               wd_ref, ws_ref, we3_ref, bcat_ref, gm_ref, bbm_ref,
               ga_ref, bba_ref, w2_ref, b2_ref, vn_ref, mo_ref):
    h = 128
    bf = jnp.bfloat16
    ef = _lnsilu(jnp.dot(eat_ref[...], we_ref[...],
                         preferred_element_type=F32) + be_ref[...],
                 ge_ref[...], bbe_ref[...])
    wd = wd_ref[...]
    ws = ws_ref[...]
    vn = vn_ref[...]
    b_eff = bcat_ref[...] + jnp.dot(vn, wd + ws, preferred_element_type=F32)
    big = (jnp.dot(gd_ref[...].astype(bf), wd.astype(bf),
                   preferred_element_type=F32)
           + jnp.dot(gs_ref[...].astype(bf), ws.astype(bf),
                     preferred_element_type=F32)
           + jnp.dot(ef.astype(bf), we3_ref[...].astype(bf),
                     preferred_element_type=F32) + b_eff)
    msg = _lnsilu(big[:, :h], gm_ref[...], bbm_ref[...])
    att = _lnsilu(big[:, h:], ga_ref[...], bba_ref[...])
    w = jnp.sum(att * w2_ref[...], axis=-1, keepdims=True) + b2_ref[...]
    mo_ref[...] = msg * _sigmoid(w)


def _tc_edge(gd, gs, edge_attr, lw, vn, blk0):
    e, h = gd.shape
    ed = edge_attr.shape[1]
    grid = e // _EDGE_BLK
    ins = [gd, gs, edge_attr,
           lw["we"], lw["be"], lw["ge"], lw["bbe"],
           lw["wd"], lw["ws"], lw["we3"], lw["bcat"],
           lw["gm"], lw["bbm"], lw["ga"], lw["bba"],
           lw["w2"], lw["b2"], vn]
    in_specs = [pl.BlockSpec((_EDGE_BLK, h), lambda i: (i, 0)),
                pl.BlockSpec((_EDGE_BLK, h), lambda i: (i, 0)),
                pl.BlockSpec((_EDGE_BLK, ed),
                             lambda i, _b=blk0: (i + _b, 0))]
    in_specs += [_full_spec(a) for a in ins[3:]]
    return pl.pallas_call(
        _edge_body,
        grid=(grid,),
        in_specs=in_specs,
        out_specs=pl.BlockSpec((_EDGE_BLK, h), lambda i: (i, 0)),
        out_shape=jax.ShapeDtypeStruct((e, h), F32),
        interpret=_INTERP,
    )(*ins)


def _node_body(parts_ref, h_ref, vn_ref, bt_ref,
               wu_ref, bu_ref, gu_ref, bbu_ref,
               wo_ref, bo_ref, go_ref, bbo_ref,
               ho_ref, vno_ref, s_acc, c_acc):
    i = pl.program_id(0)
    nb = pl.num_programs(0)
    hdim = 128

    @pl.when(i == 0)
    def _():
        s_acc[...] = jnp.zeros_like(s_acc)
        c_acc[...] = jnp.zeros_like(c_acc)
        vno_ref[...] = jnp.zeros_like(vno_ref)

    aggr = parts_ref[0] + parts_ref[1]
    x_in = h_ref[...] + vn_ref[...]
    wu = wu_ref[...]
    hu = _lnsilu(jnp.dot(aggr, wu[:hdim], preferred_element_type=F32)
                 + jnp.dot(x_in, wu[hdim:], preferred_element_type=F32)
                 + bu_ref[...], gu_ref[...], bbu_ref[...])
    xu = _lnsilu(jnp.dot(hu, wo_ref[...], preferred_element_type=F32)
                 + bo_ref[...], go_ref[...], bbo_ref[...])
    ho_ref[...] = h_ref[...] + xu

    b2 = bt_ref[0]
    rows = []
    cnts = []
    for g in range(_GBLK):
        mask = (b2 == g)
        rows.append(jnp.sum(jnp.where(mask, xu, 0.0),
                            axis=0, keepdims=True))
        cnts.append(jnp.sum(mask.astype(F32), axis=0, keepdims=True))
    s_acc[...] += jnp.concatenate(rows, axis=0)
    c_acc[...] += jnp.concatenate(cnts, axis=0)

    @pl.when(i == nb - 1)
    def _():
        cnt = c_acc[...]
        cnt = jnp.where(cnt == 0.0, 1.0, cnt)
        vng = s_acc[...] / cnt
        vno_ref[...] = jnp.mean(vng, axis=0, keepdims=True)


def _tc_node(parts, h, vn, batch3, lw):
    n, hdim = h.shape
    grid = n // _NODE_BLK
    ins = [parts, h, vn, batch3,
           lw["wu"], lw["bu"], lw["gu"], lw["bbu"],
           lw["wo"], lw["bo"], lw["go"], lw["bbo"]]
    in_specs = [pl.BlockSpec((2, _NODE_BLK, hdim), lambda i: (0, i, 0)),
                pl.BlockSpec((_NODE_BLK, hdim), lambda i: (i, 0)),
                _full_spec(vn),
                pl.BlockSpec((1, _NODE_BLK, 1), lambda i: (i, 0, 0))]
    in_specs += [_full_spec(a) for a in ins[4:]]
    return pl.pallas_call(
        _node_body,
        grid=(grid,),
        in_specs=in_specs,
        out_specs=[pl.BlockSpec((_NODE_BLK, hdim), lambda i: (i, 0)),
                   pl.BlockSpec((1, hdim), lambda i: (0, 0))],
        out_shape=[jax.ShapeDtypeStruct((n, hdim), F32),
                   jax.ShapeDtypeStruct((1, hdim), F32)],
        scratch_shapes=[pltpu.VMEM((_GBLK, hdim), F32),
                        pltpu.VMEM((_GBLK, 1), F32)],
        interpret=_INTERP,
    )(*ins)


def _pool_body(h_ref, bt_ref, wp_ref, bp_ref, gp_ref, bbp_ref, w2_ref, b2_ref,
               wm1_ref, bm1_ref, gm1_ref, bbm1_ref, wm2_ref, bm2_ref,
               wl1_ref, bl1_ref, gl1_ref, bbl1_ref, wl2_ref, bl2_ref,
               mo_ref, lo_ref, m_acc, s_acc, v_acc):
    i = pl.program_id(0)
    nb = pl.num_programs(0)

    @pl.when(i == 0)
    def _():
        m_acc[...] = jnp.full_like(m_acc, NEG_INF)
        s_acc[...] = jnp.zeros_like(s_acc)
        v_acc[...] = jnp.zeros_like(v_acc)
        mo_ref[...] = jnp.zeros_like(mo_ref)
        lo_ref[...] = jnp.zeros_like(lo_ref)

    hv = h_ref[...]
    wh = _lnsilu(jnp.dot(hv, wp_ref[...], preferred_element_type=F32)
                 + bp_ref[...], gp_ref[...], bbp_ref[...])
    w = jnp.sum(wh * w2_ref[...], axis=-1, keepdims=True) + b2_ref[...]
    b2 = bt_ref[0]
    blk = b2.shape[0]
    oh = (b2 == lax.broadcasted_iota(I32, (blk, _GBLK), 1))
    mw = jnp.where(oh, w, NEG_INF)
    bmx = jnp.max(mw, axis=0, keepdims=True)
    m_old = m_acc[...]
    m_new = jnp.maximum(m_old, bmx)
    scale = jnp.where(m_new == NEG_INF, 0.0, jnp.exp(m_old - m_new))
    nb_row = jnp.sum(jnp.where(oh, m_new, 0.0), axis=1, keepdims=True)
    e = jnp.exp(w - nb_row)
    se = jnp.sum(jnp.where(oh, e, 0.0), axis=0, keepdims=True)
    s_acc[...] = s_acc[...] * scale + se
    eh = e * hv
    rows = []
    for g in range(_GBLK):
        rows.append(jnp.sum(jnp.where(b2 == g, eh, 0.0),
                            axis=0, keepdims=True))
    vadd = jnp.concatenate(rows, axis=0)
    scale_col = jnp.concatenate(
        [scale[0, g].reshape(1, 1) for g in range(_GBLK)], axis=0)
    v_acc[...] = v_acc[...] * scale_col + vadd
    m_acc[...] = m_new

    @pl.when(i == nb - 1)
    def _():
        s = s_acc[...]
        den = jnp.where(s == 0.0, 1.0, s)
        den_col = jnp.concatenate(
            [den[0, g].reshape(1, 1) for g in range(_GBLK)], axis=0)
        gmat = v_acc[...] / den_col
        tm = _lnsilu(jnp.dot(gmat, wm1_ref[...], preferred_element_type=F32)
                     + bm1_ref[...], gm1_ref[...], bbm1_ref[...])
        mo_ref[...] = jnp.dot(tm, wm2_ref[...],
                              preferred_element_type=F32) + bm2_ref[...]
        tl = _lnsilu(jnp.dot(gmat, wl1_ref[...], preferred_element_type=F32)
                     + bl1_ref[...], gl1_ref[...], bbl1_ref[...])
        lo_ref[...] = jnp.dot(tl, wl2_ref[...],
                              preferred_element_type=F32) + bl2_ref[...]


def _tc_pool(h, batch3, pw):
    n, hdim = h.shape
    nt = pw["wm2"].shape[1]
    grid = n // _NODE_BLK
    ins = [h, batch3,
           pw["wp"], pw["bp"], pw["gp"], pw["bbp"], pw["w2"], pw["b2"],
           pw["wm1"], pw["bm1"], pw["gm1"], pw["bbm1"], pw["wm2"], pw["bm2"],
           pw["wl1"], pw["bl1"], pw["gl1"], pw["bbl1"], pw["wl2"], pw["bl2"]]
    in_specs = [pl.BlockSpec((_NODE_BLK, hdim), lambda i: (i, 0)),
                pl.BlockSpec((1, _NODE_BLK, 1), lambda i: (i, 0, 0))]
    in_specs += [_full_spec(a) for a in ins[2:]]
    return pl.pallas_call(
        _pool_body,
        grid=(grid,),
        in_specs=in_specs,
        out_specs=[pl.BlockSpec((_GBLK, nt), lambda i: (0, 0)),
                   pl.BlockSpec((_GBLK, nt), lambda i: (0, 0))],
        out_shape=[jax.ShapeDtypeStruct((_GBLK, nt), F32),
                   jax.ShapeDtypeStruct((_GBLK, nt), F32)],
        scratch_shapes=[pltpu.VMEM((1, _GBLK), F32),
                        pltpu.VMEM((1, _GBLK), F32),
                        pltpu.VMEM((_GBLK, hdim), F32)],
        interpret=_INTERP,
    )(*ins)


_GC = 80
_GP = 4
_NCHK = 5


def _sc_gather(tbl, dst4, src4):
    if _USE_JNP_SPARSE:
        return (tbl[dst4.reshape(-1)], tbl[src4.reshape(-1)])
    n, h = tbl.shape
    nch = dst4.shape[1]
    c = dst4.shape[2]
    p_depth = _GP
    epw = nch * c
    e = NW * epw
    mesh = plsc.VectorSubcoreMesh(core_axis_name="c", subcore_axis_name="s")
    sems = [pltpu.SemaphoreType.DMA for _ in range(4 * p_depth)]

    @functools.partial(
        pl.kernel, mesh=mesh,
        out_type=(jax.ShapeDtypeStruct((e, h), F32),
                  jax.ShapeDtypeStruct((e, h), F32)),
        scratch_types=[pltpu.VMEM((nch, c), I32), pltpu.VMEM((nch, c), I32),
                       pltpu.VMEM((p_depth, c, h), F32),
                       pltpu.VMEM((p_depth, c, h), F32)] + sems)
    def k(tbl_h, dst_h, src_h, gd_h, gs_h, idx_d, idx_s, bd, bs, *allsems):
        gsem_d = allsems[0:p_depth]
        gsem_s = allsems[p_depth:2 * p_depth]
        wsem_d = allsems[2 * p_depth:3 * p_depth]
        wsem_s = allsems[3 * p_depth:4 * p_depth]
        wid = lax.axis_index("s") * 2 + lax.axis_index("c")
        base = wid * epw
        pltpu.sync_copy(dst_h.at[wid], idx_d)
        pltpu.sync_copy(src_h.at[wid], idx_s)

        def gstart(p, kk):
            pltpu.async_copy(tbl_h.at[idx_d.at[kk]], bd.at[p], gsem_d[p])
            pltpu.async_copy(tbl_h.at[idx_s.at[kk]], bs.at[p], gsem_s[p])

        def gwait(p, kk):
            pltpu.make_async_copy(tbl_h.at[idx_d.at[kk]], bd.at[p],
                                  gsem_d[p]).wait()
            pltpu.make_async_copy(tbl_h.at[idx_s.at[kk]], bs.at[p],
                                  gsem_s[p]).wait()

        def wstart(p, kk):
            off = base + kk * c
            pltpu.async_copy(bd.at[p], gd_h.at[pl.ds(off, c)], wsem_d[p])
            pltpu.async_copy(bs.at[p], gs_h.at[pl.ds(off, c)], wsem_s[p])

        def wwait(p, kk):
            off = base + kk * c
            pltpu.make_async_copy(bd.at[p], gd_h.at[pl.ds(off, c)],
                                  wsem_d[p]).wait()
            pltpu.make_async_copy(bs.at[p], gs_h.at[pl.ds(off, c)],
                                  wsem_s[p]).wait()

        for p in range(p_depth):
            gstart(p, p)

        def rnd(r, carry):
            for p in range(p_depth):
                kk = r * p_depth + p
                gwait(p, kk)
                wstart(p, kk)
                nk = kk + p_depth

                @pl.when(nk < nch)
                def _():
                    wwait(p, kk)
                    gstart(p, nk)

            return carry

        rounds = nch // p_depth
        lax.fori_loop(0, rounds, rnd, 0)
        done = rounds * p_depth
        for kk in range(done, nch):
            p = kk % p_depth
            gwait(p, kk)
            wstart(p, kk)
        for kk in range(max(0, nch - p_depth), nch):
            wwait(kk % p_depth, kk)

    return k(tbl, dst4, src4)


_SP = 2


def _sc_scatter(ms, dst_sc, zeros_n):
    if _USE_JNP_SPARSE:
        m = jnp.concatenate(ms, axis=0)
        a = jax.ops.segment_sum(m, dst_sc.reshape(-1),
                                num_segments=zeros_n.shape[0])
        return jnp.stack([a, jnp.zeros_like(a)])
    nk = len(ms)
    ec, h = ms[0].shape
    n = zeros_n.shape[0]
    nch_k = dst_sc.shape[2]
    c = dst_sc.shape[3]
    p_depth = _SP
    epw = nch_k * c
    rc = 80
    nchunks = n // rc
    nrounds = (nchunks + 15) // 16
    mesh = plsc.VectorSubcoreMesh(core_axis_name="c", subcore_axis_name="s")
    sems = [pltpu.SemaphoreType.DMA for _ in range(2 * p_depth)]

    @functools.partial(
        pl.kernel, mesh=mesh,
        out_type=jax.ShapeDtypeStruct((2, n, h), F32),
        scratch_types=[pltpu.VMEM((nk * nch_k, c), I32),
                       pltpu.VMEM((p_depth, c, h), F32),
                       pltpu.VMEM_SHARED((n, h), F32)] + sems)
    def k(*refs):
        m_hs = refs[:nk]
        dst_h, z_h, out_h, idx_d, rows, acc = refs[nk:nk + 6]
        allsems = refs[nk + 6:]
        zrows = rows.at[0]
        msem = allsems[0:p_depth]
        ssem = allsems[p_depth:2 * p_depth]
        ci = lax.axis_index("c")
        si = lax.axis_index("s")
        wid = si * 2 + ci
        for kpre in range(nk):
            pltpu.sync_copy(dst_h.at[kpre, wid],
                            idx_d.at[pl.ds(kpre * nch_k, nch_k)])

        def zstep(j, carry):
            g = si + j * 16

            @pl.when(g < nchunks)
            def _():
                r0 = pl.multiple_of(g * rc, 8)
                pltpu.sync_copy(z_h.at[pl.ds(r0, rc)], zrows)
                pltpu.sync_copy(zrows, acc.at[pl.ds(r0, rc)])

            return carry

        lax.fori_loop(0, nrounds, zstep, 0)
        plsc.subcore_barrier()

        for ki in range(nk):
            m_h = m_hs[ki]
            base = wid * epw
            kbase = ki * nch_k

            def mstart(p, kk):
                off = base + kk * c
                pltpu.async_copy(m_h.at[pl.ds(off, c)], rows.at[p], msem[p])

            def mwait(p, kk):
                off = base + kk * c
                pltpu.make_async_copy(m_h.at[pl.ds(off, c)], rows.at[p],
                                      msem[p]).wait()

            def sstart(p, kk):
                pltpu.async_copy(rows.at[p], acc.at[idx_d.at[kbase + kk]],
                                 ssem[p], add=True)

            def swait(p, kk):
                pltpu.make_async_copy(rows.at[p],
                                      acc.at[idx_d.at[kbase + kk]],
                                      ssem[p]).wait()

            for p in range(p_depth):
                mstart(p, p)

            def rnd(r, carry):
                for p in range(p_depth):
                    kk = r * p_depth + p
                    mwait(p, kk)
                    sstart(p, kk)
                    nxt = kk + p_depth

                    @pl.when(nxt < nch_k)
                    def _():
                        swait(p, kk)
                        mstart(p, nxt)

                return carry

            lax.fori_loop(0, nch_k // p_depth, rnd, 0)
            done = (nch_k // p_depth) * p_depth
            for kk in range(done, nch_k):
                p = kk % p_depth
                mwait(p, kk)
                sstart(p, kk)
            for kk in range(max(0, nch_k - p_depth), nch_k):
                swait(kk % p_depth, kk)

        plsc.subcore_barrier()

        def wstep(j, carry):
            g = si + j * 16

            @pl.when(g < nchunks)
            def _():
                r0 = pl.multiple_of(g * rc, 8)
                pltpu.sync_copy(acc.at[pl.ds(r0, rc)], zrows)
                pltpu.sync_copy(zrows, out_h.at[ci, pl.ds(r0, rc)])

            return carry

        lax.fori_loop(0, nrounds, wstep, 0)

    return k(*ms, dst_sc, zeros_n)


def _row(v):
    return v.reshape(1, -1)


def _layer_weights(lp):
    h = 128
    wm = lp["msg"]["lin"]["w"]
    wa = lp["edge_att"]["lin1"]["w"]
    return {
        "we": lp["edge_mlp"]["lin"]["w"], "be": _row(lp["edge_mlp"]["lin"]["b"]),
        "ge": _row(lp["edge_mlp"]["ln"]["g"]), "bbe": _row(lp["edge_mlp"]["ln"]["b"]),
        "wd": jnp.concatenate([wm[:h], wa[:h]], axis=1),
        "ws": jnp.concatenate([wm[h:2 * h], wa[h:2 * h]], axis=1),
        "we3": jnp.concatenate([wm[2 * h:], wa[2 * h:]], axis=1),
        "bcat": jnp.concatenate([_row(lp["msg"]["lin"]["b"]),
                                 _row(lp["edge_att"]["lin1"]["b"])], axis=1),
        "gm": _row(lp["msg"]["ln"]["g"]), "bbm": _row(lp["msg"]["ln"]["b"]),
        "ga": _row(lp["edge_att"]["ln"]["g"]), "bba": _row(lp["edge_att"]["ln"]["b"]),
        "w2": _row(lp["edge_att"]["lin2"]["w"]), "b2": _row(lp["edge_att"]["lin2"]["b"]),
        "wu": lp["upd"]["lin"]["w"], "bu": _row(lp["upd"]["lin"]["b"]),
        "gu": _row(lp["upd"]["ln"]["g"]), "bbu": _row(lp["upd"]["ln"]["b"]),
        "wo": lp["out"]["lin"]["w"], "bo": _row(lp["out"]["lin"]["b"]),
        "go": _row(lp["out"]["ln"]["g"]), "bbo": _row(lp["out"]["ln"]["b"]),
    }


def _pool_weights(params):
    pa = params["pool_att"]
    om = params["out_mean"]
    ol = params["out_logvar"]
    return {
        "wp": pa["lin1"]["w"], "bp": _row(pa["lin1"]["b"]),
        "gp": _row(pa["ln"]["g"]), "bbp": _row(pa["ln"]["b"]),
        "w2": _row(pa["lin2"]["w"]), "b2": _row(pa["lin2"]["b"]),
        "wm1": om["lin1"]["w"], "bm1": _row(om["lin1"]["b"]),
        "gm1": _row(om["ln"]["g"]), "bbm1": _row(om["ln"]["b"]),
        "wm2": om["lin2"]["w"], "bm2": _row(om["lin2"]["b"]),
        "wl1": ol["lin1"]["w"], "bl1": _row(ol["lin1"]["b"]),
        "gl1": _row(ol["ln"]["g"]), "bbl1": _row(ol["ln"]["b"]),
        "wl2": ol["lin2"]["w"], "bl2": _row(ol["lin2"]["b"]),
    }


def kernel(x, edge_attr, params, edge_index, batch):
    n = x.shape[0]
    e = edge_index.shape[1]
    ec = e // _NCHK
    nch_k = ec // (NW * _GC)
    src_g = edge_index[0].reshape(_NCHK, NW, nch_k, _GC)
    dst_g = edge_index[1].reshape(_NCHK, NW, nch_k, _GC)
    batch3 = batch.reshape(n // _NODE_BLK, _NODE_BLK, 1)
    zeros_n = jnp.zeros((n, 128), F32)
    eblk_per_chunk = ec // _EDGE_BLK

    emb = params["node_emb"]
    h = _tc_embed(x, emb["w"], _row(emb["b"]))
    vn = params["virtual_node"]

    for lp in params["layers"]:
        lw = _layer_weights(lp)
        g = [_sc_gather(h, dst_g[k], src_g[k]) for k in range(_NCHK)]
        ms = [_tc_edge(g[k][0], g[k][1], edge_attr, lw, vn,
                       k * eblk_per_chunk) for k in range(_NCHK)]
        parts = _sc_scatter(ms, dst_g, zeros_n)
        h, vn = _tc_node(parts, h, vn, batch3, lw)

    mean, logvar = _tc_pool(h, batch3, _pool_weights(params))
    return (mean, logvar)


# speedup vs baseline: 1.0438x; 1.0438x over previous
import functools

import jax
import jax.numpy as jnp
from jax import lax
from jax.experimental import pallas as pl
from jax.experimental.pallas import tpu as pltpu
from jax.experimental.pallas import tpu_sc as plsc

F32 = jnp.float32
I32 = jnp.int32
NEG_INF = float("-inf")

_INTERP = False
_USE_JNP_SPARSE = False

NW = 32
_EDGE_BLK = 2560
_NODE_BLK = 1000
_GBLK = 16


def _lnsilu(y, g, b):
    mu = jnp.mean(y, axis=-1, keepdims=True)
    yc = y - mu
    var = jnp.mean(yc * yc, axis=-1, keepdims=True)
    z = yc * (lax.rsqrt(var + 1e-5) * g) + b
    zh = 0.5 * z
    return zh + zh * jnp.tanh(zh)


def _sigmoid(x):
    return 0.5 * (jnp.tanh(0.5 * x) + 1.0)


def _pack_bf(xb):
    lo = lax.bitcast_convert_type(xb[:, :64], jnp.uint16).astype(jnp.uint32)
    hi = lax.bitcast_convert_type(xb[:, 64:], jnp.uint16).astype(jnp.uint32)
    return lax.bitcast_convert_type(lo | (hi << 16), jnp.int32)


def _unpack_bf(xi):
    u = lax.bitcast_convert_type(xi, jnp.uint32)
    lo = lax.bitcast_convert_type((u & 0xFFFF).astype(jnp.uint16),
                                  jnp.bfloat16)
    hi = lax.bitcast_convert_type(
        lax.shift_right_logical(u, jnp.uint32(16)).astype(jnp.uint16),
        jnp.bfloat16)
    return jnp.concatenate([lo, hi], axis=1)


def _full_spec(arr):
    nd = arr.ndim
    return pl.BlockSpec(arr.shape, lambda i, _nd=nd: (0,) * _nd)


def _embed_body(x_ref, w_ref, b_ref, o_ref):
    o_ref[...] = jnp.dot(x_ref[...], w_ref[...],
                         preferred_element_type=F32) + b_ref[...]


def _tc_embed(x, w, b):
    n, d = x.shape
    h = w.shape[1]
    grid = n // _NODE_BLK
    return pl.pallas_call(
        _embed_body,
        grid=(grid,),
        in_specs=[pl.BlockSpec((_NODE_BLK, d), lambda i: (i, 0)),
                  _full_spec(w), _full_spec(b)],
        out_specs=pl.BlockSpec((_NODE_BLK, h), lambda i: (i, 0)),
        out_shape=jax.ShapeDtypeStruct((n, h), F32),
        interpret=_INTERP,
    )(x, w, b)


def _edge_body(gd_ref, gs_ref, eat_ref, we_ref, be_ref, ge_ref, bbe_ref,
               wd_ref, ws_ref, we3_ref, bcat_ref, gm_ref, bbm_ref,
               ga_ref, bba_ref, w2_ref, b2_ref, vn_ref, mo_ref):
    h = 128
    bf = jnp.bfloat16
    ef_lin = lax.dot_general(eat_ref[...], we_ref[...],
                             (((0,), (0,)), ((), ())),
                             preferred_element_type=F32)
    ef = _lnsilu(ef_lin + be_ref[...], ge_ref[...], bbe_ref[...])
    wd = wd_ref[...]
    ws = ws_ref[...]
    vn = vn_ref[...]
    b_eff = bcat_ref[...] + jnp.dot(vn, wd + ws, preferred_element_type=F32)
    big = (jnp.dot(gd_ref[...].astype(bf), wd.astype(bf),
                   preferred_element_type=F32)
           + jnp.dot(gs_ref[...].astype(bf), ws.astype(bf),
                     preferred_element_type=F32)
           + jnp.dot(ef.astype(bf), we3_ref[...].astype(bf),
                     preferred_element_type=F32) + b_eff)
    msg = _lnsilu(big[:, :h], gm_ref[...], bbm_ref[...])
    att = _lnsilu(big[:, h:], ga_ref[...], bba_ref[...])
    w = jnp.sum(att * w2_ref[...], axis=-1, keepdims=True) + b2_ref[...]
    mo_ref[...] = msg * _sigmoid(w)


def _tc_edge(gd, gs, eat_t, lw, vn, blk0):
    e, h = gd.shape
    ed = eat_t.shape[0]
    grid = e // _EDGE_BLK
    ins = [gd, gs, eat_t,
           lw["we"], lw["be"], lw["ge"], lw["bbe"],
           lw["wd"], lw["ws"], lw["we3"], lw["bcat"],
           lw["gm"], lw["bbm"], lw["ga"], lw["bba"],
           lw["w2"], lw["b2"], vn]
    in_specs = [pl.BlockSpec((_EDGE_BLK, h), lambda i: (i, 0)),
                pl.BlockSpec((_EDGE_BLK, h), lambda i: (i, 0)),
                pl.BlockSpec((ed, _EDGE_BLK),
                             lambda i, _b=blk0: (0, i + _b))]
    in_specs += [_full_spec(a) for a in ins[3:]]
    return pl.pallas_call(
        _edge_body,
        grid=(grid,),
        in_specs=in_specs,
        out_specs=pl.BlockSpec((_EDGE_BLK, h), lambda i: (i, 0)),
        out_shape=jax.ShapeDtypeStruct((e, h), F32),
        interpret=_INTERP,
    )(*ins)


def _node_body(parts_ref, h_ref, vn_ref, bt_ref,
               wu_ref, bu_ref, gu_ref, bbu_ref,
               wo_ref, bo_ref, go_ref, bbo_ref,
               ho_ref, vno_ref, s_acc, c_acc):
    i = pl.program_id(0)
    nb = pl.num_programs(0)
    hdim = 128

    @pl.when(i == 0)
    def _():
        s_acc[...] = jnp.zeros_like(s_acc)
        c_acc[...] = jnp.zeros_like(c_acc)
        vno_ref[...] = jnp.zeros_like(vno_ref)

    aggr = parts_ref[0] + parts_ref[1]
    x_in = h_ref[...] + vn_ref[...]
    wu = wu_ref[...]
    hu = _lnsilu(jnp.dot(aggr, wu[:hdim], preferred_element_type=F32)
                 + jnp.dot(x_in, wu[hdim:], preferred_element_type=F32)
                 + bu_ref[...], gu_ref[...], bbu_ref[...])
    xu = _lnsilu(jnp.dot(hu, wo_ref[...], preferred_element_type=F32)
                 + bo_ref[...], go_ref[...], bbo_ref[...])
    ho_ref[...] = h_ref[...] + xu

    b2 = bt_ref[0]
    rows = []
    cnts = []
    for g in range(_GBLK):
        mask = (b2 == g)
        rows.append(jnp.sum(jnp.where(mask, xu, 0.0),
                            axis=0, keepdims=True))
        cnts.append(jnp.sum(mask.astype(F32), axis=0, keepdims=True))
    s_acc[...] += jnp.concatenate(rows, axis=0)
    c_acc[...] += jnp.concatenate(cnts, axis=0)

    @pl.when(i == nb - 1)
    def _():
        cnt = c_acc[...]
        cnt = jnp.where(cnt == 0.0, 1.0, cnt)
        vng = s_acc[...] / cnt
        vno_ref[...] = jnp.mean(vng, axis=0, keepdims=True)


def _tc_node(parts, h, vn, batch3, lw):
    n, hdim = h.shape
    grid = n // _NODE_BLK
    ins = [parts, h, vn, batch3,
           lw["wu"], lw["bu"], lw["gu"], lw["bbu"],
           lw["wo"], lw["bo"], lw["go"], lw["bbo"]]
    in_specs = [pl.BlockSpec((2, _NODE_BLK, hdim), lambda i: (0, i, 0)),
                pl.BlockSpec((_NODE_BLK, hdim), lambda i: (i, 0)),
                _full_spec(vn),
                pl.BlockSpec((1, _NODE_BLK, 1), lambda i: (i, 0, 0))]
    in_specs += [_full_spec(a) for a in ins[4:]]
    return pl.pallas_call(
        _node_body,
        grid=(grid,),
        in_specs=in_specs,
        out_specs=[pl.BlockSpec((_NODE_BLK, hdim), lambda i: (i, 0)),
                   pl.BlockSpec((1, hdim), lambda i: (0, 0))],
        out_shape=[jax.ShapeDtypeStruct((n, hdim), F32),
                   jax.ShapeDtypeStruct((1, hdim), F32)],
        scratch_shapes=[pltpu.VMEM((_GBLK, hdim), F32),
                        pltpu.VMEM((_GBLK, 1), F32)],
        interpret=_INTERP,
    )(*ins)


def _pool_body(h_ref, bt_ref, wp_ref, bp_ref, gp_ref, bbp_ref, w2_ref, b2_ref,
               wm1_ref, bm1_ref, gm1_ref, bbm1_ref, wm2_ref, bm2_ref,
               wl1_ref, bl1_ref, gl1_ref, bbl1_ref, wl2_ref, bl2_ref,
               mo_ref, lo_ref, m_acc, s_acc, v_acc):
    i = pl.program_id(0)
    nb = pl.num_programs(0)

    @pl.when(i == 0)
    def _():
        m_acc[...] = jnp.full_like(m_acc, NEG_INF)
        s_acc[...] = jnp.zeros_like(s_acc)
        v_acc[...] = jnp.zeros_like(v_acc)
        mo_ref[...] = jnp.zeros_like(mo_ref)
        lo_ref[...] = jnp.zeros_like(lo_ref)

    hv = h_ref[...]
    wh = _lnsilu(jnp.dot(hv, wp_ref[...], preferred_element_type=F32)
                 + bp_ref[...], gp_ref[...], bbp_ref[...])
    w = jnp.sum(wh * w2_ref[...], axis=-1, keepdims=True) + b2_ref[...]
    b2 = bt_ref[0]
    blk = b2.shape[0]
    oh = (b2 == lax.broadcasted_iota(I32, (blk, _GBLK), 1))
    mw = jnp.where(oh, w, NEG_INF)
    bmx = jnp.max(mw, axis=0, keepdims=True)
    m_old = m_acc[...]
    m_new = jnp.maximum(m_old, bmx)
    scale = jnp.where(m_new == NEG_INF, 0.0, jnp.exp(m_old - m_new))
    nb_row = jnp.sum(jnp.where(oh, m_new, 0.0), axis=1, keepdims=True)
    e = jnp.exp(w - nb_row)
    se = jnp.sum(jnp.where(oh, e, 0.0), axis=0, keepdims=True)
    s_acc[...] = s_acc[...] * scale + se
    eh = e * hv
    rows = []
    for g in range(_GBLK):
        rows.append(jnp.sum(jnp.where(b2 == g, eh, 0.0),
                            axis=0, keepdims=True))
    vadd = jnp.concatenate(rows, axis=0)
    scale_col = jnp.concatenate(
        [scale[0, g].reshape(1, 1) for g in range(_GBLK)], axis=0)
    v_acc[...] = v_acc[...] * scale_col + vadd
    m_acc[...] = m_new

    @pl.when(i == nb - 1)
    def _():
        s = s_acc[...]
        den = jnp.where(s == 0.0, 1.0, s)
        den_col = jnp.concatenate(
            [den[0, g].reshape(1, 1) for g in range(_GBLK)], axis=0)
        gmat = v_acc[...] / den_col
        tm = _lnsilu(jnp.dot(gmat, wm1_ref[...], preferred_element_type=F32)
                     + bm1_ref[...], gm1_ref[...], bbm1_ref[...])
        mo_ref[...] = jnp.dot(tm, wm2_ref[...],
                              preferred_element_type=F32) + bm2_ref[...]
        tl = _lnsilu(jnp.dot(gmat, wl1_ref[...], preferred_element_type=F32)
                     + bl1_ref[...], gl1_ref[...], bbl1_ref[...])
        lo_ref[...] = jnp.dot(tl, wl2_ref[...],
                              preferred_element_type=F32) + bl2_ref[...]


def _tc_pool(h, batch3, pw):
    n, hdim = h.shape
    nt = pw["wm2"].shape[1]
    grid = n // _NODE_BLK
    ins = [h, batch3,
           pw["wp"], pw["bp"], pw["gp"], pw["bbp"], pw["w2"], pw["b2"],
           pw["wm1"], pw["bm1"], pw["gm1"], pw["bbm1"], pw["wm2"], pw["bm2"],
           pw["wl1"], pw["bl1"], pw["gl1"], pw["bbl1"], pw["wl2"], pw["bl2"]]
    in_specs = [pl.BlockSpec((_NODE_BLK, hdim), lambda i: (i, 0)),
                pl.BlockSpec((1, _NODE_BLK, 1), lambda i: (i, 0, 0))]
    in_specs += [_full_spec(a) for a in ins[2:]]
    return pl.pallas_call(
        _pool_body,
        grid=(grid,),
        in_specs=in_specs,
        out_specs=[pl.BlockSpec((_GBLK, nt), lambda i: (0, 0)),
                   pl.BlockSpec((_GBLK, nt), lambda i: (0, 0))],
        out_shape=[jax.ShapeDtypeStruct((_GBLK, nt), F32),
                   jax.ShapeDtypeStruct((_GBLK, nt), F32)],
        scratch_shapes=[pltpu.VMEM((1, _GBLK), F32),
                        pltpu.VMEM((1, _GBLK), F32),
                        pltpu.VMEM((_GBLK, hdim), F32)],
        interpret=_INTERP,
    )(*ins)


_GC = 80
_GP = 4
_NCHK = 5


def _sc_gather(tbl, dst4, src4):
    if _USE_JNP_SPARSE:
        return (tbl[dst4.reshape(-1)], tbl[src4.reshape(-1)])
    n, h = tbl.shape
    nch = dst4.shape[1]
    c = dst4.shape[2]
    p_depth = _GP
    epw = nch * c
    e = NW * epw
    mesh = plsc.VectorSubcoreMesh(core_axis_name="c", subcore_axis_name="s")
    sems = [pltpu.SemaphoreType.DMA for _ in range(4 * p_depth)]

    @functools.partial(
        pl.kernel, mesh=mesh,
        out_type=(jax.ShapeDtypeStruct((e, h), F32),
                  jax.ShapeDtypeStruct((e, h), F32)),
        scratch_types=[pltpu.VMEM((nch, c), I32), pltpu.VMEM((nch, c), I32),
                       pltpu.VMEM((p_depth, c, h), F32),
                       pltpu.VMEM((p_depth, c, h), F32)] + sems)
    def k(tbl_h, dst_h, src_h, gd_h, gs_h, idx_d, idx_s, bd, bs, *allsems):
        gsem_d = allsems[0:p_depth]
        gsem_s = allsems[p_depth:2 * p_depth]
        wsem_d = allsems[2 * p_depth:3 * p_depth]
        wsem_s = allsems[3 * p_depth:4 * p_depth]
        wid = lax.axis_index("s") * 2 + lax.axis_index("c")
        base = wid * epw
        pltpu.sync_copy(dst_h.at[wid], idx_d)
        pltpu.sync_copy(src_h.at[wid], idx_s)

        def gstart(p, kk):
            pltpu.async_copy(tbl_h.at[idx_d.at[kk]], bd.at[p], gsem_d[p])
            pltpu.async_copy(tbl_h.at[idx_s.at[kk]], bs.at[p], gsem_s[p])

        def gwait(p, kk):
            pltpu.make_async_copy(tbl_h.at[idx_d.at[kk]], bd.at[p],
                                  gsem_d[p]).wait()
            pltpu.make_async_copy(tbl_h.at[idx_s.at[kk]], bs.at[p],
                                  gsem_s[p]).wait()

        def wstart(p, kk):
            off = base + kk * c
            pltpu.async_copy(bd.at[p], gd_h.at[pl.ds(off, c)], wsem_d[p])
            pltpu.async_copy(bs.at[p], gs_h.at[pl.ds(off, c)], wsem_s[p])

        def wwait(p, kk):
            off = base + kk * c
            pltpu.make_async_copy(bd.at[p], gd_h.at[pl.ds(off, c)],
                                  wsem_d[p]).wait()
            pltpu.make_async_copy(bs.at[p], gs_h.at[pl.ds(off, c)],
                                  wsem_s[p]).wait()

        for p in range(p_depth):
            gstart(p, p)

        def rnd(r, carry):
            for p in range(p_depth):
                kk = r * p_depth + p
                gwait(p, kk)
                wstart(p, kk)
                nk = kk + p_depth

                @pl.when(nk < nch)
                def _():
                    wwait(p, kk)
                    gstart(p, nk)

            return carry

        rounds = nch // p_depth
        lax.fori_loop(0, rounds, rnd, 0)
        done = rounds * p_depth
        for kk in range(done, nch):
            p = kk % p_depth
            gwait(p, kk)
            wstart(p, kk)
        for kk in range(max(0, nch - p_depth), nch):
            wwait(kk % p_depth, kk)

    return k(tbl, dst4, src4)


_SP = 2


def _sc_scatter(ms, dst_sc, zeros_n):
    if _USE_JNP_SPARSE:
        m = jnp.concatenate(ms, axis=0)
        a = jax.ops.segment_sum(m, dst_sc.reshape(-1),
                                num_segments=zeros_n.shape[0])
        return jnp.stack([a, jnp.zeros_like(a)])
    nk = len(ms)
    ec, h = ms[0].shape
    n = zeros_n.shape[0]
    nch_k = dst_sc.shape[2]
    c = dst_sc.shape[3]
    p_depth = _SP
    epw = nch_k * c
    rc = 80
    nchunks = n // rc
    nrounds = (nchunks + 15) // 16
    mesh = plsc.VectorSubcoreMesh(core_axis_name="c", subcore_axis_name="s")
    sems = [pltpu.SemaphoreType.DMA for _ in range(2 * p_depth)]

    @functools.partial(
        pl.kernel, mesh=mesh,
        out_type=jax.ShapeDtypeStruct((2, n, h), F32),
        scratch_types=[pltpu.VMEM((nk * nch_k, c), I32),
                       pltpu.VMEM((p_depth, c, h), F32),
                       pltpu.VMEM_SHARED((n, h), F32)] + sems)
    def k(*refs):
        m_hs = refs[:nk]
        dst_h, z_h, out_h, idx_d, rows, acc = refs[nk:nk + 6]
        allsems = refs[nk + 6:]
        zrows = rows.at[0]
        msem = allsems[0:p_depth]
        ssem = allsems[p_depth:2 * p_depth]
        ci = lax.axis_index("c")
        si = lax.axis_index("s")
        wid = si * 2 + ci
        for kpre in range(nk):
            pltpu.sync_copy(dst_h.at[kpre, wid],
                            idx_d.at[pl.ds(kpre * nch_k, nch_k)])

        def zstep(j, carry):
            g = si + j * 16

            @pl.when(g < nchunks)
            def _():
                r0 = pl.multiple_of(g * rc, 8)
                pltpu.sync_copy(z_h.at[pl.ds(r0, rc)], zrows)
                pltpu.sync_copy(zrows, acc.at[pl.ds(r0, rc)])

            return carry

        lax.fori_loop(0, nrounds, zstep, 0)
        plsc.subcore_barrier()

        for ki in range(nk):
            m_h = m_hs[ki]
            base = wid * epw
            kbase = ki * nch_k

            def mstart(p, kk):
                off = base + kk * c
                pltpu.async_copy(m_h.at[pl.ds(off, c)], rows.at[p], msem[p])

            def mwait(p, kk):
                off = base + kk * c
                pltpu.make_async_copy(m_h.at[pl.ds(off, c)], rows.at[p],
                                      msem[p]).wait()

            def sstart(p, kk):
                pltpu.async_copy(rows.at[p], acc.at[idx_d.at[kbase + kk]],
                                 ssem[p], add=True)

            def swait(p, kk):
                pltpu.make_async_copy(rows.at[p],
                                      acc.at[idx_d.at[kbase + kk]],
                                      ssem[p]).wait()

            for p in range(p_depth):
                mstart(p, p)

            def rnd(r, carry):
                for p in range(p_depth):
                    kk = r * p_depth + p
                    mwait(p, kk)
                    sstart(p, kk)
                    nxt = kk + p_depth

                    @pl.when(nxt < nch_k)
                    def _():
                        swait(p, kk)
                        mstart(p, nxt)

                return carry

            lax.fori_loop(0, nch_k // p_depth, rnd, 0)
            done = (nch_k // p_depth) * p_depth
            for kk in range(done, nch_k):
                p = kk % p_depth
                mwait(p, kk)
                sstart(p, kk)
            for kk in range(max(0, nch_k - p_depth), nch_k):
                swait(kk % p_depth, kk)

        plsc.subcore_barrier()

        def wstep(j, carry):
            g = si + j * 16

            @pl.when(g < nchunks)
            def _():
                r0 = pl.multiple_of(g * rc, 8)
                pltpu.sync_copy(acc.at[pl.ds(r0, rc)], zrows)
                pltpu.sync_copy(zrows, out_h.at[ci, pl.ds(r0, rc)])

            return carry

        lax.fori_loop(0, nrounds, wstep, 0)

    return k(*ms, dst_sc, zeros_n)


def _row(v):
    return v.reshape(1, -1)


def _layer_weights(lp):
    h = 128
    wm = lp["msg"]["lin"]["w"]
    wa = lp["edge_att"]["lin1"]["w"]
    return {
        "we": lp["edge_mlp"]["lin"]["w"], "be": _row(lp["edge_mlp"]["lin"]["b"]),
        "ge": _row(lp["edge_mlp"]["ln"]["g"]), "bbe": _row(lp["edge_mlp"]["ln"]["b"]),
        "wd": jnp.concatenate([wm[:h], wa[:h]], axis=1),
        "ws": jnp.concatenate([wm[h:2 * h], wa[h:2 * h]], axis=1),
        "we3": jnp.concatenate([wm[2 * h:], wa[2 * h:]], axis=1),
        "bcat": jnp.concatenate([_row(lp["msg"]["lin"]["b"]),
                                 _row(lp["edge_att"]["lin1"]["b"])], axis=1),
        "gm": _row(lp["msg"]["ln"]["g"]), "bbm": _row(lp["msg"]["ln"]["b"]),
        "ga": _row(lp["edge_att"]["ln"]["g"]), "bba": _row(lp["edge_att"]["ln"]["b"]),
        "w2": _row(lp["edge_att"]["lin2"]["w"]), "b2": _row(lp["edge_att"]["lin2"]["b"]),
        "wu": lp["upd"]["lin"]["w"], "bu": _row(lp["upd"]["lin"]["b"]),
        "gu": _row(lp["upd"]["ln"]["g"]), "bbu": _row(lp["upd"]["ln"]["b"]),
        "wo": lp["out"]["lin"]["w"], "bo": _row(lp["out"]["lin"]["b"]),
        "go": _row(lp["out"]["ln"]["g"]), "bbo": _row(lp["out"]["ln"]["b"]),
    }


def _pool_weights(params):
    pa = params["pool_att"]
    om = params["out_mean"]
    ol = params["out_logvar"]
    return {
        "wp": pa["lin1"]["w"], "bp": _row(pa["lin1"]["b"]),
        "gp": _row(pa["ln"]["g"]), "bbp": _row(pa["ln"]["b"]),
        "w2": _row(pa["lin2"]["w"]), "b2": _row(pa["lin2"]["b"]),
        "wm1": om["lin1"]["w"], "bm1": _row(om["lin1"]["b"]),
        "gm1": _row(om["ln"]["g"]), "bbm1": _row(om["ln"]["b"]),
        "wm2": om["lin2"]["w"], "bm2": _row(om["lin2"]["b"]),
        "wl1": ol["lin1"]["w"], "bl1": _row(ol["lin1"]["b"]),
        "gl1": _row(ol["ln"]["g"]), "bbl1": _row(ol["ln"]["b"]),
        "wl2": ol["lin2"]["w"], "bl2": _row(ol["lin2"]["b"]),
    }


def kernel(x, edge_attr, params, edge_index, batch):
    n = x.shape[0]
    e = edge_index.shape[1]
    ec = e // _NCHK
    nch_k = ec // (NW * _GC)
    src_g = edge_index[0].reshape(_NCHK, NW, nch_k, _GC)
    dst_g = edge_index[1].reshape(_NCHK, NW, nch_k, _GC)
    batch3 = batch.reshape(n // _NODE_BLK, _NODE_BLK, 1)
    zeros_n = jnp.zeros((n, 128), F32)
    eat_t = edge_attr.T
    eblk_per_chunk = ec // _EDGE_BLK

    emb = params["node_emb"]
    h = _tc_embed(x, emb["w"], _row(emb["b"]))
    vn = params["virtual_node"]

    for lp in params["layers"]:
        lw = _layer_weights(lp)
        g = [_sc_gather(h, dst_g[k], src_g[k]) for k in range(_NCHK)]
        ms = [_tc_edge(g[k][0], g[k][1], eat_t, lw, vn,
                       k * eblk_per_chunk) for k in range(_NCHK)]
        parts = _sc_scatter(ms, dst_g, zeros_n)
        h, vn = _tc_node(parts, h, vn, batch3, lw)

    mean, logvar = _tc_pool(h, batch3, _pool_weights(params))
    return (mean, logvar)


# speedup vs baseline: 1.0445x; 1.0006x over previous
import functools

import jax
import jax.numpy as jnp
from jax import lax
from jax.experimental import pallas as pl
from jax.experimental.pallas import tpu as pltpu
from jax.experimental.pallas import tpu_sc as plsc

F32 = jnp.float32
I32 = jnp.int32
NEG_INF = float("-inf")

NW = 32
_EDGE_BLK = 2560
_NODE_BLK = 1000
_GBLK = 16


def _lnsilu(y, g, b):
    mu = jnp.mean(y, axis=-1, keepdims=True)
    yc = y - mu
    var = jnp.mean(yc * yc, axis=-1, keepdims=True)
    z = yc * (lax.rsqrt(var + 1e-5) * g) + b
    zh = 0.5 * z
    return zh + zh * jnp.tanh(zh)


def _sigmoid(x):
    return 0.5 * (jnp.tanh(0.5 * x) + 1.0)


def _full_spec(arr):
    nd = arr.ndim
    return pl.BlockSpec(arr.shape, lambda i, _nd=nd: (0,) * _nd)


def _embed_body(x_ref, w_ref, b_ref, o_ref):
    o_ref[...] = jnp.dot(x_ref[...], w_ref[...],
                         preferred_element_type=F32) + b_ref[...]


def _tc_embed(x, w, b):
    n, d = x.shape
    h = w.shape[1]
    grid = n // _NODE_BLK
    return pl.pallas_call(
        _embed_body,
        grid=(grid,),
        in_specs=[pl.BlockSpec((_NODE_BLK, d), lambda i: (i, 0)),
                  _full_spec(w), _full_spec(b)],
        out_specs=pl.BlockSpec((_NODE_BLK, h), lambda i: (i, 0)),
        out_shape=jax.ShapeDtypeStruct((n, h), F32),
    )(x, w, b)


def _edge_body(gd_ref, gs_ref, eat_ref, we_ref, be_ref, ge_ref, bbe_ref,
               wd_ref, ws_ref, we3_ref, bcat_ref, gm_ref, bbm_ref,
               ga_ref, bba_ref, w2_ref, b2_ref, vn_ref, mo_ref):
    h = 128
    bf = jnp.bfloat16
    ef_lin = lax.dot_general(eat_ref[...], we_ref[...],
                             (((0,), (0,)), ((), ())),
                             preferred_element_type=F32)
    ef = _lnsilu(ef_lin + be_ref[...], ge_ref[...], bbe_ref[...])
    wd = wd_ref[...]
    ws = ws_ref[...]
    vn = vn_ref[...]
    b_eff = bcat_ref[...] + jnp.dot(vn, wd + ws, preferred_element_type=F32)
    big = (jnp.dot(gd_ref[...].astype(bf), wd.astype(bf),
                   preferred_element_type=F32)
           + jnp.dot(gs_ref[...].astype(bf), ws.astype(bf),
                     preferred_element_type=F32)
           + jnp.dot(ef.astype(bf), we3_ref[...].astype(bf),
                     preferred_element_type=F32) + b_eff)
    msg = _lnsilu(big[:, :h], gm_ref[...], bbm_ref[...])
    att = _lnsilu(big[:, h:], ga_ref[...], bba_ref[...])
    w = jnp.sum(att * w2_ref[...], axis=-1, keepdims=True) + b2_ref[...]
    mo_ref[...] = msg * _sigmoid(w)


def _tc_edge(gd, gs, eat_t, lw, vn, blk0):
    e, h = gd.shape
    ed = eat_t.shape[0]
    grid = e // _EDGE_BLK
    ins = [gd, gs, eat_t,
           lw["we"], lw["be"], lw["ge"], lw["bbe"],
           lw["wd"], lw["ws"], lw["we3"], lw["bcat"],
           lw["gm"], lw["bbm"], lw["ga"], lw["bba"],
           lw["w2"], lw["b2"], vn]
    in_specs = [pl.BlockSpec((_EDGE_BLK, h), lambda i: (i, 0)),
                pl.BlockSpec((_EDGE_BLK, h), lambda i: (i, 0)),
                pl.BlockSpec((ed, _EDGE_BLK),
                             lambda i, _b=blk0: (0, i + _b))]
    in_specs += [_full_spec(a) for a in ins[3:]]
    return pl.pallas_call(
        _edge_body,
        grid=(grid,),
        in_specs=in_specs,
        out_specs=pl.BlockSpec((_EDGE_BLK, h), lambda i: (i, 0)),
        out_shape=jax.ShapeDtypeStruct((e, h), F32),
    )(*ins)


def _node_body(parts_ref, h_ref, vn_ref, bt_ref,
               wu_ref, bu_ref, gu_ref, bbu_ref,
               wo_ref, bo_ref, go_ref, bbo_ref,
               ho_ref, vno_ref, s_acc, c_acc):
    i = pl.program_id(0)
    nb = pl.num_programs(0)
    hdim = 128

    @pl.when(i == 0)
    def _():
        s_acc[...] = jnp.zeros_like(s_acc)
        c_acc[...] = jnp.zeros_like(c_acc)
        vno_ref[...] = jnp.zeros_like(vno_ref)

    aggr = parts_ref[0] + parts_ref[1]
    x_in = h_ref[...] + vn_ref[...]
    wu = wu_ref[...]
    hu = _lnsilu(jnp.dot(aggr, wu[:hdim], preferred_element_type=F32)
                 + jnp.dot(x_in, wu[hdim:], preferred_element_type=F32)
                 + bu_ref[...], gu_ref[...], bbu_ref[...])
    xu = _lnsilu(jnp.dot(hu, wo_ref[...], preferred_element_type=F32)
                 + bo_ref[...], go_ref[...], bbo_ref[...])
    ho_ref[...] = h_ref[...] + xu

    b2 = bt_ref[0]
    rows = []
    cnts = []
    for g in range(_GBLK):
        mask = (b2 == g)
        rows.append(jnp.sum(jnp.where(mask, xu, 0.0),
                            axis=0, keepdims=True))
        cnts.append(jnp.sum(mask.astype(F32), axis=0, keepdims=True))
    s_acc[...] += jnp.concatenate(rows, axis=0)
    c_acc[...] += jnp.concatenate(cnts, axis=0)

    @pl.when(i == nb - 1)
    def _():
        cnt = c_acc[...]
        cnt = jnp.where(cnt == 0.0, 1.0, cnt)
        vng = s_acc[...] / cnt
        vno_ref[...] = jnp.mean(vng, axis=0, keepdims=True)


def _tc_node(parts, h, vn, batch3, lw):
    n, hdim = h.shape
    grid = n // _NODE_BLK
    ins = [parts, h, vn, batch3,
           lw["wu"], lw["bu"], lw["gu"], lw["bbu"],
           lw["wo"], lw["bo"], lw["go"], lw["bbo"]]
    in_specs = [pl.BlockSpec((2, _NODE_BLK, hdim), lambda i: (0, i, 0)),
                pl.BlockSpec((_NODE_BLK, hdim), lambda i: (i, 0)),
                _full_spec(vn),
                pl.BlockSpec((1, _NODE_BLK, 1), lambda i: (i, 0, 0))]
    in_specs += [_full_spec(a) for a in ins[4:]]
    return pl.pallas_call(
        _node_body,
        grid=(grid,),
        in_specs=in_specs,
        out_specs=[pl.BlockSpec((_NODE_BLK, hdim), lambda i: (i, 0)),
                   pl.BlockSpec((1, hdim), lambda i: (0, 0))],
        out_shape=[jax.ShapeDtypeStruct((n, hdim), F32),
                   jax.ShapeDtypeStruct((1, hdim), F32)],
        scratch_shapes=[pltpu.VMEM((_GBLK, hdim), F32),
                        pltpu.VMEM((_GBLK, 1), F32)],
    )(*ins)


def _pool_body(h_ref, bt_ref, wp_ref, bp_ref, gp_ref, bbp_ref, w2_ref, b2_ref,
               wm1_ref, bm1_ref, gm1_ref, bbm1_ref, wm2_ref, bm2_ref,
               wl1_ref, bl1_ref, gl1_ref, bbl1_ref, wl2_ref, bl2_ref,
               mo_ref, lo_ref, m_acc, s_acc, v_acc):
    i = pl.program_id(0)
    nb = pl.num_programs(0)

    @pl.when(i == 0)
    def _():
        m_acc[...] = jnp.full_like(m_acc, NEG_INF)
        s_acc[...] = jnp.zeros_like(s_acc)
        v_acc[...] = jnp.zeros_like(v_acc)
        mo_ref[...] = jnp.zeros_like(mo_ref)
        lo_ref[...] = jnp.zeros_like(lo_ref)

    hv = h_ref[...]
    wh = _lnsilu(jnp.dot(hv, wp_ref[...], preferred_element_type=F32)
                 + bp_ref[...], gp_ref[...], bbp_ref[...])
    w = jnp.sum(wh * w2_ref[...], axis=-1, keepdims=True) + b2_ref[...]
    b2 = bt_ref[0]
    blk = b2.shape[0]
    oh = (b2 == lax.broadcasted_iota(I32, (blk, _GBLK), 1))
    mw = jnp.where(oh, w, NEG_INF)
    bmx = jnp.max(mw, axis=0, keepdims=True)
    m_old = m_acc[...]
    m_new = jnp.maximum(m_old, bmx)
    scale = jnp.where(m_new == NEG_INF, 0.0, jnp.exp(m_old - m_new))
    nb_row = jnp.sum(jnp.where(oh, m_new, 0.0), axis=1, keepdims=True)
    e = jnp.exp(w - nb_row)
    se = jnp.sum(jnp.where(oh, e, 0.0), axis=0, keepdims=True)
    s_acc[...] = s_acc[...] * scale + se
    eh = e * hv
    rows = []
    for g in range(_GBLK):
        rows.append(jnp.sum(jnp.where(b2 == g, eh, 0.0),
                            axis=0, keepdims=True))
    vadd = jnp.concatenate(rows, axis=0)
    scale_col = jnp.concatenate(
        [scale[0, g].reshape(1, 1) for g in range(_GBLK)], axis=0)
    v_acc[...] = v_acc[...] * scale_col + vadd
    m_acc[...] = m_new

    @pl.when(i == nb - 1)
    def _():
        s = s_acc[...]
        den = jnp.where(s == 0.0, 1.0, s)
        den_col = jnp.concatenate(
            [den[0, g].reshape(1, 1) for g in range(_GBLK)], axis=0)
        gmat = v_acc[...] / den_col
        tm = _lnsilu(jnp.dot(gmat, wm1_ref[...], preferred_element_type=F32)
                     + bm1_ref[...], gm1_ref[...], bbm1_ref[...])
        mo_ref[...] = jnp.dot(tm, wm2_ref[...],
                              preferred_element_type=F32) + bm2_ref[...]
        tl = _lnsilu(jnp.dot(gmat, wl1_ref[...], preferred_element_type=F32)
                     + bl1_ref[...], gl1_ref[...], bbl1_ref[...])
        lo_ref[...] = jnp.dot(tl, wl2_ref[...],
                              preferred_element_type=F32) + bl2_ref[...]


def _tc_pool(h, batch3, pw):
    n, hdim = h.shape
    nt = pw["wm2"].shape[1]
    grid = n // _NODE_BLK
    ins = [h, batch3,
           pw["wp"], pw["bp"], pw["gp"], pw["bbp"], pw["w2"], pw["b2"],
           pw["wm1"], pw["bm1"], pw["gm1"], pw["bbm1"], pw["wm2"], pw["bm2"],
           pw["wl1"], pw["bl1"], pw["gl1"], pw["bbl1"], pw["wl2"], pw["bl2"]]
    in_specs = [pl.BlockSpec((_NODE_BLK, hdim), lambda i: (i, 0)),
                pl.BlockSpec((1, _NODE_BLK, 1), lambda i: (i, 0, 0))]
    in_specs += [_full_spec(a) for a in ins[2:]]
    return pl.pallas_call(
        _pool_body,
        grid=(grid,),
        in_specs=in_specs,
        out_specs=[pl.BlockSpec((_GBLK, nt), lambda i: (0, 0)),
                   pl.BlockSpec((_GBLK, nt), lambda i: (0, 0))],
        out_shape=[jax.ShapeDtypeStruct((_GBLK, nt), F32),
                   jax.ShapeDtypeStruct((_GBLK, nt), F32)],
        scratch_shapes=[pltpu.VMEM((1, _GBLK), F32),
                        pltpu.VMEM((1, _GBLK), F32),
                        pltpu.VMEM((_GBLK, hdim), F32)],
    )(*ins)


_GC = 80
_GP = 4
_NCHK = 5


def _sc_gather(tbl, dst4, src4):
    n, h = tbl.shape
    nch = dst4.shape[1]
    c = dst4.shape[2]
    p_depth = _GP
    epw = nch * c
    e = NW * epw
    mesh = plsc.VectorSubcoreMesh(core_axis_name="c", subcore_axis_name="s")
    sems = [pltpu.SemaphoreType.DMA for _ in range(4 * p_depth)]

    @functools.partial(
        pl.kernel, mesh=mesh,
        out_type=(jax.ShapeDtypeStruct((e, h), F32),
                  jax.ShapeDtypeStruct((e, h), F32)),
        scratch_types=[pltpu.VMEM((nch, c), I32), pltpu.VMEM((nch, c), I32),
                       pltpu.VMEM((p_depth, c, h), F32),
                       pltpu.VMEM((p_depth, c, h), F32)] + sems)
    def k(tbl_h, dst_h, src_h, gd_h, gs_h, idx_d, idx_s, bd, bs, *allsems):
        gsem_d = allsems[0:p_depth]
        gsem_s = allsems[p_depth:2 * p_depth]
        wsem_d = allsems[2 * p_depth:3 * p_depth]
        wsem_s = allsems[3 * p_depth:4 * p_depth]
        wid = lax.axis_index("s") * 2 + lax.axis_index("c")
        base = wid * epw
        pltpu.sync_copy(dst_h.at[wid], idx_d)
        pltpu.sync_copy(src_h.at[wid], idx_s)

        def gstart(p, kk):
            pltpu.async_copy(tbl_h.at[idx_d.at[kk]], bd.at[p], gsem_d[p])
            pltpu.async_copy(tbl_h.at[idx_s.at[kk]], bs.at[p], gsem_s[p])

        def gwait(p, kk):
            pltpu.make_async_copy(tbl_h.at[idx_d.at[kk]], bd.at[p],
                                  gsem_d[p]).wait()
            pltpu.make_async_copy(tbl_h.at[idx_s.at[kk]], bs.at[p],
                                  gsem_s[p]).wait()

        def wstart(p, kk):
            off = base + kk * c
            pltpu.async_copy(bd.at[p], gd_h.at[pl.ds(off, c)], wsem_d[p])
            pltpu.async_copy(bs.at[p], gs_h.at[pl.ds(off, c)], wsem_s[p])

        def wwait(p, kk):
            off = base + kk * c
            pltpu.make_async_copy(bd.at[p], gd_h.at[pl.ds(off, c)],
                                  wsem_d[p]).wait()
            pltpu.make_async_copy(bs.at[p], gs_h.at[pl.ds(off, c)],
                                  wsem_s[p]).wait()

        for p in range(p_depth):
            gstart(p, p)

        def rnd(r, carry):
            for p in range(p_depth):
                kk = r * p_depth + p
                gwait(p, kk)
                wstart(p, kk)
                nk = kk + p_depth

                @pl.when(nk < nch)
                def _():
                    wwait(p, kk)
                    gstart(p, nk)

            return carry

        rounds = nch // p_depth
        lax.fori_loop(0, rounds, rnd, 0)
        done = rounds * p_depth
        for kk in range(done, nch):
            p = kk % p_depth
            gwait(p, kk)
            wstart(p, kk)
        for kk in range(max(0, nch - p_depth), nch):
            wwait(kk % p_depth, kk)

    return k(tbl, dst4, src4)


_SP = 2


def _sc_scatter(ms, dst_sc, zeros_n):
    nk = len(ms)
    ec, h = ms[0].shape
    n = zeros_n.shape[0]
    nch_k = dst_sc.shape[2]
    c = dst_sc.shape[3]
    p_depth = _SP
    epw = nch_k * c
    rc = 80
    nchunks = n // rc
    nrounds = (nchunks + 15) // 16
    mesh = plsc.VectorSubcoreMesh(core_axis_name="c", subcore_axis_name="s")
    sems = [pltpu.SemaphoreType.DMA for _ in range(2 * p_depth)]

    @functools.partial(
        pl.kernel, mesh=mesh,
        out_type=jax.ShapeDtypeStruct((2, n, h), F32),
        scratch_types=[pltpu.VMEM((nk * nch_k, c), I32),
                       pltpu.VMEM((p_depth, c, h), F32),
                       pltpu.VMEM_SHARED((n, h), F32)] + sems)
    def k(*refs):
        m_hs = refs[:nk]
        dst_h, z_h, out_h, idx_d, rows, acc = refs[nk:nk + 6]
        allsems = refs[nk + 6:]
        zrows = rows.at[0]
        msem = allsems[0:p_depth]
        ssem = allsems[p_depth:2 * p_depth]
        ci = lax.axis_index("c")
        si = lax.axis_index("s")
        wid = si * 2 + ci
        for kpre in range(nk):
            pltpu.sync_copy(dst_h.at[kpre, wid],
                            idx_d.at[pl.ds(kpre * nch_k, nch_k)])

        def zstep(j, carry):
            g = si + j * 16

            @pl.when(g < nchunks)
            def _():
                r0 = pl.multiple_of(g * rc, 8)
                pltpu.sync_copy(z_h.at[pl.ds(r0, rc)], zrows)
                pltpu.sync_copy(zrows, acc.at[pl.ds(r0, rc)])

            return carry

        lax.fori_loop(0, nrounds, zstep, 0)
        plsc.subcore_barrier()

        for ki in range(nk):
            m_h = m_hs[ki]
            base = wid * epw
            kbase = ki * nch_k

            def mstart(p, kk):
                off = base + kk * c
                pltpu.async_copy(m_h.at[pl.ds(off, c)], rows.at[p], msem[p])

            def mwait(p, kk):
                off = base + kk * c
                pltpu.make_async_copy(m_h.at[pl.ds(off, c)], rows.at[p],
                                      msem[p]).wait()

            def sstart(p, kk):
                pltpu.async_copy(rows.at[p], acc.at[idx_d.at[kbase + kk]],
                                 ssem[p], add=True)

            def swait(p, kk):
                pltpu.make_async_copy(rows.at[p],
                                      acc.at[idx_d.at[kbase + kk]],
                                      ssem[p]).wait()

            for p in range(p_depth):
                mstart(p, p)

            def rnd(r, carry):
                for p in range(p_depth):
                    kk = r * p_depth + p
                    mwait(p, kk)
                    sstart(p, kk)
                    nxt = kk + p_depth

                    @pl.when(nxt < nch_k)
                    def _():
                        swait(p, kk)
                        mstart(p, nxt)

                return carry

            lax.fori_loop(0, nch_k // p_depth, rnd, 0)
            done = (nch_k // p_depth) * p_depth
            for kk in range(done, nch_k):
                p = kk % p_depth
                mwait(p, kk)
                sstart(p, kk)
            for kk in range(max(0, nch_k - p_depth), nch_k):
                swait(kk % p_depth, kk)

        plsc.subcore_barrier()

        def wstep(j, carry):
            g = si + j * 16

            @pl.when(g < nchunks)
            def _():
                r0 = pl.multiple_of(g * rc, 8)
                pltpu.sync_copy(acc.at[pl.ds(r0, rc)], zrows)
                pltpu.sync_copy(zrows, out_h.at[ci, pl.ds(r0, rc)])

            return carry

        lax.fori_loop(0, nrounds, wstep, 0)

    return k(*ms, dst_sc, zeros_n)


def _row(v):
    return v.reshape(1, -1)


def _layer_weights(lp):
    h = 128
    wm = lp["msg"]["lin"]["w"]
    wa = lp["edge_att"]["lin1"]["w"]
    return {
        "we": lp["edge_mlp"]["lin"]["w"], "be": _row(lp["edge_mlp"]["lin"]["b"]),
        "ge": _row(lp["edge_mlp"]["ln"]["g"]), "bbe": _row(lp["edge_mlp"]["ln"]["b"]),
        "wd": jnp.concatenate([wm[:h], wa[:h]], axis=1),
        "ws": jnp.concatenate([wm[h:2 * h], wa[h:2 * h]], axis=1),
        "we3": jnp.concatenate([wm[2 * h:], wa[2 * h:]], axis=1),
        "bcat": jnp.concatenate([_row(lp["msg"]["lin"]["b"]),
                                 _row(lp["edge_att"]["lin1"]["b"])], axis=1),
        "gm": _row(lp["msg"]["ln"]["g"]), "bbm": _row(lp["msg"]["ln"]["b"]),
        "ga": _row(lp["edge_att"]["ln"]["g"]), "bba": _row(lp["edge_att"]["ln"]["b"]),
        "w2": _row(lp["edge_att"]["lin2"]["w"]), "b2": _row(lp["edge_att"]["lin2"]["b"]),
        "wu": lp["upd"]["lin"]["w"], "bu": _row(lp["upd"]["lin"]["b"]),
        "gu": _row(lp["upd"]["ln"]["g"]), "bbu": _row(lp["upd"]["ln"]["b"]),
        "wo": lp["out"]["lin"]["w"], "bo": _row(lp["out"]["lin"]["b"]),
        "go": _row(lp["out"]["ln"]["g"]), "bbo": _row(lp["out"]["ln"]["b"]),
    }


def _pool_weights(params):
    pa = params["pool_att"]
    om = params["out_mean"]
    ol = params["out_logvar"]
    return {
        "wp": pa["lin1"]["w"], "bp": _row(pa["lin1"]["b"]),
        "gp": _row(pa["ln"]["g"]), "bbp": _row(pa["ln"]["b"]),
        "w2": _row(pa["lin2"]["w"]), "b2": _row(pa["lin2"]["b"]),
        "wm1": om["lin1"]["w"], "bm1": _row(om["lin1"]["b"]),
        "gm1": _row(om["ln"]["g"]), "bbm1": _row(om["ln"]["b"]),
        "wm2": om["lin2"]["w"], "bm2": _row(om["lin2"]["b"]),
        "wl1": ol["lin1"]["w"], "bl1": _row(ol["lin1"]["b"]),
        "gl1": _row(ol["ln"]["g"]), "bbl1": _row(ol["ln"]["b"]),
        "wl2": ol["lin2"]["w"], "bl2": _row(ol["lin2"]["b"]),
    }


def kernel(x, edge_attr, params, edge_index, batch):
    n = x.shape[0]
    e = edge_index.shape[1]
    ec = e // _NCHK
    nch_k = ec // (NW * _GC)
    src_g = edge_index[0].reshape(_NCHK, NW, nch_k, _GC)
    dst_g = edge_index[1].reshape(_NCHK, NW, nch_k, _GC)
    batch3 = batch.reshape(n // _NODE_BLK, _NODE_BLK, 1)
    zeros_n = jnp.zeros((n, 128), F32)
    eat_t = edge_attr.T
    eblk_per_chunk = ec // _EDGE_BLK

    emb = params["node_emb"]
    h = _tc_embed(x, emb["w"], _row(emb["b"]))
    vn = params["virtual_node"]

    for lp in params["layers"]:
        lw = _layer_weights(lp)
        g = [_sc_gather(h, dst_g[k], src_g[k]) for k in range(_NCHK)]
        ms = [_tc_edge(g[k][0], g[k][1], eat_t, lw, vn,
                       k * eblk_per_chunk) for k in range(_NCHK)]
        parts = _sc_scatter(ms, dst_g, zeros_n)
        h, vn = _tc_node(parts, h, vn, batch3, lw)

    mean, logvar = _tc_pool(h, batch3, _pool_weights(params))
    return (mean, logvar)
